# Initial kernel scaffold; baseline (speedup 1.0000x reference)
import jax
import jax.numpy as jnp
from jax.experimental import pallas as pl


def kernel(x, edge_index, W0, b0, W1, b1):
    raise NotImplementedError("write your pallas kernel here")


# speedup vs baseline: 8.4447x; 8.4447x over previous
import functools

import jax
import jax.numpy as jnp
from jax import lax
from jax.experimental import pallas as pl
from jax.experimental.pallas import tpu as pltpu
from jax.experimental.pallas import tpu_sc as plsc

_N = 10000
_E = 320000
_DG = 64
_DW = 16
_NC = 2
_NS = 16
_NW = _NC * _NS
_CH = 80
_EPW = _E // _NW
_CPW = _EPW // _CH
_RPS = 624
_TAIL0 = _RPS * _NS
_TAILN = _N - _TAIL0


def _make_sc_agg(compute_deg):
  mesh = plsc.VectorSubcoreMesh(core_axis_name="c", subcore_axis_name="s")
  out_type = [jax.ShapeDtypeStruct((_NC, _N, _DG), jnp.float32)]
  scratch = [
      pltpu.VMEM((_CPW, _CH), jnp.int32),
      pltpu.VMEM((_CPW, _CH), jnp.int32),
      pltpu.VMEM((_CH, _DG), jnp.float32),
      pltpu.VMEM_SHARED((_N, _DG), jnp.float32),
  ]
  if compute_deg:
    out_type.append(jax.ShapeDtypeStruct((_NC, _N, _DW), jnp.float32))
    scratch += [
        pltpu.VMEM((_CH, _DW), jnp.float32),
        pltpu.VMEM_SHARED((_N, _DW), jnp.float32),
    ]

  def body(*refs):
    if compute_deg:
      (vals, src, dst, z64, zdw, ones_h, out_sum, out_deg,
       srcb, dstb, rows, acc, onesb, dacc) = refs
    else:
      (vals, src, dst, z64, out_sum,
       srcb, dstb, rows, acc) = refs
    cid = lax.axis_index("c")
    sid = lax.axis_index("s")
    wid = sid * _NC + cid
    r0 = sid * _RPS
    pltpu.sync_copy(z64.at[pl.ds(r0, _RPS)], acc.at[pl.ds(r0, _RPS)])
    if compute_deg:
      pltpu.sync_copy(zdw.at[pl.ds(r0, _RPS)], dacc.at[pl.ds(r0, _RPS)])
      pltpu.sync_copy(ones_h, onesb)

    @pl.when(sid == _NS - 1)
    def _():
      pltpu.sync_copy(z64.at[pl.ds(_TAIL0, _TAILN)],
                      acc.at[pl.ds(_TAIL0, _TAILN)])
      if compute_deg:
        pltpu.sync_copy(zdw.at[pl.ds(_TAIL0, _TAILN)],
                        dacc.at[pl.ds(_TAIL0, _TAILN)])
    pltpu.sync_copy(src.at[wid], srcb)
    pltpu.sync_copy(dst.at[wid], dstb)
    plsc.subcore_barrier()

    @pl.loop(0, _CPW)
    def _(c):
      pltpu.sync_copy(vals.at[srcb.at[c]], rows)
      pltpu.sync_copy(rows, acc.at[dstb.at[c]], add=True)
      if compute_deg:
        pltpu.sync_copy(onesb, dacc.at[dstb.at[c]], add=True)

    plsc.subcore_barrier()
    pltpu.sync_copy(acc.at[pl.ds(r0, _RPS)],
                    out_sum.at[cid].at[pl.ds(r0, _RPS)])
    if compute_deg:
      pltpu.sync_copy(dacc.at[pl.ds(r0, _RPS)],
                      out_deg.at[cid].at[pl.ds(r0, _RPS)])

    @pl.when(sid == _NS - 1)
    def _():
      pltpu.sync_copy(acc.at[pl.ds(_TAIL0, _TAILN)],
                      out_sum.at[cid].at[pl.ds(_TAIL0, _TAILN)])
      if compute_deg:
        pltpu.sync_copy(dacc.at[pl.ds(_TAIL0, _TAILN)],
                        out_deg.at[cid].at[pl.ds(_TAIL0, _TAILN)])

  return pl.kernel(
      body,
      out_type=tuple(out_type) if compute_deg else out_type[0],
      mesh=mesh,
      scratch_types=scratch,
      compiler_params=pltpu.CompilerParams(use_tc_tiling_on_sc=False),
  )


_sc_agg_deg = _make_sc_agg(True)
_sc_agg = _make_sc_agg(False)

_R = 1000


def _tc_conv1_body(sa, sb, da, db, xb, wb, bb, ob):
  s = sa[...] + sb[...]
  deg = da[...] + db[...]
  inv = 1.0 / jnp.maximum(deg[:, 0:1], 1.0)
  agg = jnp.dot(s * inv, wb[...], preferred_element_type=jnp.float32)
  ob[...] = xb[...] + agg + bb[...]


def _tc_conv1(sa, sb, da, db, x0, W0, b0):
  grid = (_N // _R,)
  row = lambda i: (i, 0)
  fixed = lambda i: (0, 0)
  return pl.pallas_call(
      _tc_conv1_body,
      grid=grid,
      in_specs=[
          pl.BlockSpec((_R, _DG), row),
          pl.BlockSpec((_R, _DG), row),
          pl.BlockSpec((_R, _DW), row),
          pl.BlockSpec((_R, _DW), row),
          pl.BlockSpec((_R, _DG), row),
          pl.BlockSpec((_DG, _DG), fixed),
          pl.BlockSpec((1, _DG), fixed),
      ],
      out_specs=pl.BlockSpec((_R, _DG), row),
      out_shape=jax.ShapeDtypeStruct((_N, _DG), jnp.float32),
  )(sa, sb, da, db, x0, W0, b0)


def _tc_conv2_body(sa, sb, da, db, xb, y0b, wb, bb, ob):
  s = sa[...] + sb[...]
  deg = da[...] + db[...]
  inv = 1.0 / jnp.maximum(deg[:, 0:1], 1.0)
  agg = jnp.dot(s * inv, wb[...], preferred_element_type=jnp.float32)
  y1 = xb[...] + agg + bb[...]
  ob[...] = jnp.concatenate([y0b[...], y1], axis=1)


def _tc_conv2(sa, sb, da, db, x1, y0, W1, b1):
  grid = (_N // _R,)
  row = lambda i: (i, 0)
  fixed = lambda i: (0, 0)
  return pl.pallas_call(
      _tc_conv2_body,
      grid=grid,
      in_specs=[
          pl.BlockSpec((_R, _DG), row),
          pl.BlockSpec((_R, _DG), row),
          pl.BlockSpec((_R, _DW), row),
          pl.BlockSpec((_R, _DW), row),
          pl.BlockSpec((_R, _DG), row),
          pl.BlockSpec((_R, _DG), row),
          pl.BlockSpec((_DG, _DG), fixed),
          pl.BlockSpec((1, _DG), fixed),
      ],
      out_specs=pl.BlockSpec((_R, 2 * _DG), row),
      out_shape=jax.ShapeDtypeStruct((_N, 2 * _DG), jnp.float32),
  )(sa, sb, da, db, x1, y0, W1, b1)


def kernel(x, edge_index, W0, b0, W1, b1):
  src = edge_index[0].reshape(_NW, _CPW, _CH)
  dst = edge_index[1].reshape(_NW, _CPW, _CH)
  x0 = x[:, :_DG]
  x1 = x[:, _DG:]
  z64 = jnp.zeros((_N, _DG), jnp.float32)
  zdw = jnp.zeros((_N, _DW), jnp.float32)
  ones = jnp.ones((_CH, _DW), jnp.float32)
  b0r = b0.reshape(1, _DG)
  b1r = b1.reshape(1, _DG)

  s0, degp = _sc_agg_deg(x1, src, dst, z64, zdw, ones)
  y0 = _tc_conv1(s0[0], s0[1], degp[0], degp[1], x0, W0, b0r)
  s1 = _sc_agg(y0, src, dst, z64)
  return _tc_conv2(s1[0], s1[1], degp[0], degp[1], x1, y0, W1, b1r)


# speedup vs baseline: 14.8966x; 1.7640x over previous
import functools

import jax
import jax.numpy as jnp
from jax import lax
from jax.experimental import pallas as pl
from jax.experimental.pallas import tpu as pltpu
from jax.experimental.pallas import tpu_sc as plsc

_N = 10000
_E = 320000
_DG = 64
_DW = 16
_NC = 2
_NS = 16
_NW = _NC * _NS
_CH = 80
_EPW = _E // _NW
_CPW = _EPW // _CH
_RPS = 624
_TAIL0 = _RPS * _NS
_TAILN = _N - _TAIL0
_NBUF = 5
_P = 3


def _make_sc_agg(compute_deg):
  mesh = plsc.VectorSubcoreMesh(core_axis_name="c", subcore_axis_name="s")
  out_type = [jax.ShapeDtypeStruct((_NC, _N, _DG), jnp.float32)]
  scratch = [
      pltpu.VMEM((_CPW, _CH), jnp.int32),
      pltpu.VMEM((_CPW, _CH), jnp.int32),
      pltpu.VMEM((_NBUF, _CH, _DG), jnp.float32),
      pltpu.VMEM_SHARED((_N, _DG), jnp.float32),
      pltpu.SemaphoreType.DMA((_NBUF,)),
      pltpu.SemaphoreType.DMA((_NBUF,)),
  ]
  if compute_deg:
    out_type.append(jax.ShapeDtypeStruct((_NC, _N, _DW), jnp.float32))
    scratch += [
        pltpu.VMEM((_CH, _DW), jnp.float32),
        pltpu.VMEM_SHARED((_N, _DW), jnp.float32),
        pltpu.SemaphoreType.DMA((_NBUF,)),
    ]

  def body(*refs):
    if compute_deg:
      (vals, src, dst, z64, zdw, ones_h, out_sum, out_deg,
       srcb, dstb, rows, acc, gsem, ssem, onesb, dacc, dsem) = refs
    else:
      (vals, src, dst, z64, out_sum,
       srcb, dstb, rows, acc, gsem, ssem) = refs
    cid = lax.axis_index("c")
    sid = lax.axis_index("s")
    wid = sid * _NC + cid
    r0 = sid * _RPS
    pltpu.sync_copy(z64.at[pl.ds(r0, _RPS)], acc.at[pl.ds(r0, _RPS)])
    if compute_deg:
      pltpu.sync_copy(zdw.at[pl.ds(r0, _RPS)], dacc.at[pl.ds(r0, _RPS)])
      pltpu.sync_copy(ones_h, onesb)

    @pl.when(sid == _NS - 1)
    def _():
      pltpu.sync_copy(z64.at[pl.ds(_TAIL0, _TAILN)],
                      acc.at[pl.ds(_TAIL0, _TAILN)])
      if compute_deg:
        pltpu.sync_copy(zdw.at[pl.ds(_TAIL0, _TAILN)],
                        dacc.at[pl.ds(_TAIL0, _TAILN)])
    pltpu.sync_copy(src.at[wid], srcb)
    pltpu.sync_copy(dst.at[wid], dstb)
    plsc.subcore_barrier()

    for b in range(_P):
      pltpu.async_copy(vals.at[srcb.at[b]], rows.at[b], gsem.at[b])

    @pl.loop(0, _CPW, step=_NBUF)
    def _(c):
      for b in range(_NBUF):
        cc = c + b
        pltpu.make_async_copy(vals.at[srcb.at[cc]], rows.at[b],
                              gsem.at[b]).wait()
        pltpu.async_copy(rows.at[b], acc.at[dstb.at[cc]], ssem.at[b],
                         add=True)
        if compute_deg:
          @pl.when(cc >= _NBUF)
          def _():
            pltpu.make_async_copy(onesb, dacc.at[dstb.at[cc]],
                                  dsem.at[b]).wait()
          pltpu.async_copy(onesb, dacc.at[dstb.at[cc]], dsem.at[b],
                           add=True)
        bn = (b + _P) % _NBUF

        @pl.when(jnp.logical_and(cc + _P < _CPW, cc + _P >= _NBUF))
        def _():
          pltpu.make_async_copy(rows.at[bn], acc.at[dstb.at[cc]],
                                ssem.at[bn]).wait()

        @pl.when(cc + _P < _CPW)
        def _():
          pltpu.async_copy(vals.at[srcb.at[cc + _P]], rows.at[bn],
                           gsem.at[bn])

    for b in range(_NBUF):
      pltpu.make_async_copy(rows.at[b], acc.at[dstb.at[0]],
                            ssem.at[b]).wait()
      if compute_deg:
        pltpu.make_async_copy(onesb, dacc.at[dstb.at[0]],
                              dsem.at[b]).wait()

    plsc.subcore_barrier()
    pltpu.sync_copy(acc.at[pl.ds(r0, _RPS)],
                    out_sum.at[cid].at[pl.ds(r0, _RPS)])
    if compute_deg:
      pltpu.sync_copy(dacc.at[pl.ds(r0, _RPS)],
                      out_deg.at[cid].at[pl.ds(r0, _RPS)])

    @pl.when(sid == _NS - 1)
    def _():
      pltpu.sync_copy(acc.at[pl.ds(_TAIL0, _TAILN)],
                      out_sum.at[cid].at[pl.ds(_TAIL0, _TAILN)])
      if compute_deg:
        pltpu.sync_copy(dacc.at[pl.ds(_TAIL0, _TAILN)],
                        out_deg.at[cid].at[pl.ds(_TAIL0, _TAILN)])

  return pl.kernel(
      body,
      out_type=tuple(out_type) if compute_deg else out_type[0],
      mesh=mesh,
      scratch_types=scratch,
      compiler_params=pltpu.CompilerParams(use_tc_tiling_on_sc=False),
  )


_sc_agg_deg = _make_sc_agg(True)
_sc_agg = _make_sc_agg(False)

_R = 1000


def _tc_conv1_body(sa, sb, da, db, xb, wb, bb, ob):
  s = sa[...] + sb[...]
  deg = da[...] + db[...]
  inv = 1.0 / jnp.maximum(deg[:, 0:1], 1.0)
  agg = jnp.dot(s * inv, wb[...], preferred_element_type=jnp.float32)
  ob[...] = xb[...] + agg + bb[...]


def _tc_conv1(sa, sb, da, db, x0, W0, b0):
  grid = (_N // _R,)
  row = lambda i: (i, 0)
  fixed = lambda i: (0, 0)
  return pl.pallas_call(
      _tc_conv1_body,
      grid=grid,
      in_specs=[
          pl.BlockSpec((_R, _DG), row),
          pl.BlockSpec((_R, _DG), row),
          pl.BlockSpec((_R, _DW), row),
          pl.BlockSpec((_R, _DW), row),
          pl.BlockSpec((_R, _DG), row),
          pl.BlockSpec((_DG, _DG), fixed),
          pl.BlockSpec((1, _DG), fixed),
      ],
      out_specs=pl.BlockSpec((_R, _DG), row),
      out_shape=jax.ShapeDtypeStruct((_N, _DG), jnp.float32),
  )(sa, sb, da, db, x0, W0, b0)


def _tc_conv2_body(sa, sb, da, db, xb, y0b, wb, bb, ob):
  s = sa[...] + sb[...]
  deg = da[...] + db[...]
  inv = 1.0 / jnp.maximum(deg[:, 0:1], 1.0)
  agg = jnp.dot(s * inv, wb[...], preferred_element_type=jnp.float32)
  y1 = xb[...] + agg + bb[...]
  ob[...] = jnp.concatenate([y0b[...], y1], axis=1)


def _tc_conv2(sa, sb, da, db, x1, y0, W1, b1):
  grid = (_N // _R,)
  row = lambda i: (i, 0)
  fixed = lambda i: (0, 0)
  return pl.pallas_call(
      _tc_conv2_body,
      grid=grid,
      in_specs=[
          pl.BlockSpec((_R, _DG), row),
          pl.BlockSpec((_R, _DG), row),
          pl.BlockSpec((_R, _DW), row),
          pl.BlockSpec((_R, _DW), row),
          pl.BlockSpec((_R, _DG), row),
          pl.BlockSpec((_R, _DG), row),
          pl.BlockSpec((_DG, _DG), fixed),
          pl.BlockSpec((1, _DG), fixed),
      ],
      out_specs=pl.BlockSpec((_R, 2 * _DG), row),
      out_shape=jax.ShapeDtypeStruct((_N, 2 * _DG), jnp.float32),
  )(sa, sb, da, db, x1, y0, W1, b1)


def kernel(x, edge_index, W0, b0, W1, b1):
  src = edge_index[0].reshape(_NW, _CPW, _CH)
  dst = edge_index[1].reshape(_NW, _CPW, _CH)
  x0 = x[:, :_DG]
  x1 = x[:, _DG:]
  z64 = jnp.zeros((_N, _DG), jnp.float32)
  zdw = jnp.zeros((_N, _DW), jnp.float32)
  ones = jnp.ones((_CH, _DW), jnp.float32)
  b0r = b0.reshape(1, _DG)
  b1r = b1.reshape(1, _DG)

  s0, degp = _sc_agg_deg(x1, src, dst, z64, zdw, ones)
  y0 = _tc_conv1(s0[0], s0[1], degp[0], degp[1], x0, W0, b0r)
  s1 = _sc_agg(y0, src, dst, z64)
  return _tc_conv2(s1[0], s1[1], degp[0], degp[1], x1, y0, W1, b1r)
